# baseline (speedup 1.0000x reference)
import functools

import jax
import jax.numpy as jnp
from jax.experimental import pallas as pl


def _xw_kernel(x_ref, w_ref, o_ref):
    o_ref[:, :] = jnp.dot(x_ref[:, :], w_ref[:, :],
                          preferred_element_type=jnp.float32)


def _pass1_kernel(adj_ref, g_ref, b1_ref, w2_ref, p_ref):
    h = jnp.dot(adj_ref[:, :], g_ref[:, :],
                preferred_element_type=jnp.float32)
    h = jnp.maximum(h + b1_ref[:], 0.0)
    p_ref[:, :] = jnp.dot(h, w2_ref[:, :],
                          preferred_element_type=jnp.float32)


def _pass2_kernel(adj_ref, p_ref, b2_ref, o_ref):
    o = jnp.dot(adj_ref[:, :], p_ref[:, :],
                preferred_element_type=jnp.float32)
    o = o + b2_ref[:]
    m = jnp.max(o, axis=1, keepdims=True)
    e = o - m
    lse = jnp.log(jnp.sum(jnp.exp(e), axis=1, keepdims=True))
    o_ref[:, :] = e - lse


@jax.jit
def _run(x, adj, W1, b1, W2, b2):
    N, nfeat = x.shape
    nhid = W1.shape[1]
    nclass = W2.shape[1]

    g = pl.pallas_call(
        _xw_kernel,
        out_shape=jax.ShapeDtypeStruct((N, nhid), jnp.float32),
    )(x, W1)

    BM = 400
    grid = (N // BM,)

    p = pl.pallas_call(
        _pass1_kernel,
        grid=grid,
        in_specs=[
            pl.BlockSpec((BM, N), lambda i: (i, 0)),
            pl.BlockSpec((N, nhid), lambda i: (0, 0)),
            pl.BlockSpec((nhid,), lambda i: (0,)),
            pl.BlockSpec((nhid, nclass), lambda i: (0, 0)),
        ],
        out_specs=pl.BlockSpec((BM, nclass), lambda i: (i, 0)),
        out_shape=jax.ShapeDtypeStruct((N, nclass), jnp.float32),
    )(adj, g, b1, W2)

    out = pl.pallas_call(
        _pass2_kernel,
        grid=grid,
        in_specs=[
            pl.BlockSpec((BM, N), lambda i: (i, 0)),
            pl.BlockSpec((N, nclass), lambda i: (0, 0)),
            pl.BlockSpec((nclass,), lambda i: (0,)),
        ],
        out_specs=pl.BlockSpec((BM, nclass), lambda i: (i, 0)),
        out_shape=jax.ShapeDtypeStruct((N, nclass), jnp.float32),
    )(adj, p, b2)

    return out


def kernel(x, adj, W1, b1, W2, b2, epoch, test):
    del epoch, test
    return _run(x, adj, W1, b1, W2, b2)
